# Initial kernel scaffold; baseline (speedup 1.0000x reference)
import jax
import jax.numpy as jnp
from jax.experimental import pallas as pl


def kernel(input_labels, out_labels, noise_indices, node_emb, ctx_emb):
    raise NotImplementedError("write your pallas kernel here")


# speedup vs baseline: 1.0587x; 1.0587x over previous
import functools

import jax
import jax.numpy as jnp
import numpy as np
from jax import lax
from jax.experimental import pallas as pl
from jax.experimental.pallas import tpu as pltpu
from jax.experimental.pallas import tpu_sc as plsc

VOCAB = 1000000
DIM = 64
BATCH = 16384
WINDOW = 20
NEG = 5
BW = BATCH * WINDOW
PAD = 8

NW = 32
WIN_PER_SUB = BATCH // NW
CHUNK_WIN = 8
CHUNK_PAIRS = CHUNK_WIN * WINDOW
CHUNK_NOISE = CHUNK_PAIRS * NEG
NCHUNK = WIN_PER_SUB // CHUNK_WIN
GROUPS = CHUNK_PAIRS // 16


def _sc_dots(node_emb, ctx_emb, inp_idx2d, out_idx2d, noise_idx2d, winrow):
    mesh = plsc.VectorSubcoreMesh(core_axis_name="c", subcore_axis_name="s")

    @functools.partial(
        pl.kernel,
        out_type=jax.ShapeDtypeStruct((BW * PAD,), jnp.float32),
        mesh=mesh,
        compiler_params=pltpu.CompilerParams(needs_layout_passes=False,
                                             use_tc_tiling_on_sc=False),
        scratch_types=[
            pltpu.VMEM((CHUNK_WIN,), jnp.int32),
            pltpu.VMEM((2, 80), jnp.int32),
            pltpu.VMEM((8, 100), jnp.int32),
            pltpu.VMEM((CHUNK_PAIRS,), jnp.int32),
            pltpu.VMEM((CHUNK_WIN, DIM), jnp.float32),
            pltpu.VMEM((CHUNK_PAIRS, DIM), jnp.float32),
            pltpu.VMEM((CHUNK_NOISE, DIM), jnp.float32),
            pltpu.VMEM((CHUNK_PAIRS * PAD,), jnp.float32),
            pltpu.SemaphoreType.DMA,
        ],
    )
    def k(node_hbm, ctx_hbm, inp_hbm, out_hbm, noise_hbm, winrow_hbm,
          dots_hbm, inp_idx_v, out_idx_v, noise_idx_v, winrow_v,
          inp_rows, out_rows, noise_rows, dots_v, sem):
        wid = lax.axis_index("s") * 2 + lax.axis_index("c")
        pltpu.sync_copy(winrow_hbm, winrow_v)
        lam = lax.iota(jnp.int32, 16)

        def chunk_body(c, carry):
            row = wid * NCHUNK + c
            pltpu.sync_copy(inp_hbm.at[row], inp_idx_v)
            pltpu.sync_copy(out_hbm.at[pl.ds(row * 2, 2)], out_idx_v)
            pltpu.sync_copy(noise_hbm.at[pl.ds(row * 8, 8)], noise_idx_v)

            handles = [pltpu.async_copy(node_hbm.at[inp_idx_v], inp_rows,
                                        sem)]
            for j in range(2):
                handles.append(pltpu.async_copy(
                    ctx_hbm.at[out_idx_v.at[j]],
                    out_rows.at[pl.ds(j * 80, 80)], sem))
            for j in range(8):
                handles.append(pltpu.async_copy(
                    ctx_hbm.at[noise_idx_v.at[j]],
                    noise_rows.at[pl.ds(j * 100, 100)], sem))
            for h in handles:
                h.wait()

            def group_body(g, carry2):
                pair16 = g * 16 + lam
                win_v = winrow_v[pl.ds(g * 16, 16)]
                pair5 = pair16 * NEG
                accs = [jnp.zeros((16,), jnp.float32) for _ in range(6)]
                for d in range(DIM):
                    dv = jnp.full((16,), d, jnp.int32)
                    a = plsc.load_gather(inp_rows, [win_v, dv])
                    o = plsc.load_gather(out_rows, [pair16, dv])
                    accs[0] = accs[0] + a * o
                    for n in range(NEG):
                        x = plsc.load_gather(noise_rows, [pair5 + n, dv])
                        accs[1 + n] = accs[1 + n] - a * x
                base8 = pair16 * PAD
                for t in range(6):
                    plsc.store_scatter(dots_v, [base8 + t], accs[t])
                return carry2

            lax.fori_loop(0, GROUPS, group_body, 0)
            pltpu.sync_copy(
                dots_v,
                dots_hbm.at[pl.ds(row * (CHUNK_PAIRS * PAD),
                                  CHUNK_PAIRS * PAD)])
            return carry

        lax.fori_loop(0, NCHUNK, chunk_body, 0)

    return k(node_emb, ctx_emb, inp_idx2d, out_idx2d, noise_idx2d, winrow)


_TC_ROWS = 2560
_TC_COLS = 1024
_TC_BLK = 256
_TC_GRID = _TC_ROWS // _TC_BLK


def _tc_reduce_body(x_ref, o_ref):
    i = pl.program_id(0)
    x = x_ref[...]
    z = jnp.minimum(x, 0.0) - jnp.log1p(jnp.exp(-jnp.abs(x)))
    col = lax.broadcasted_iota(jnp.int32, (_TC_BLK, _TC_COLS), 1)
    z = jnp.where((col % PAD) < 6, z, 0.0)
    s = jnp.sum(z)

    @pl.when(i == 0)
    def _():
        o_ref[0, 0] = 0.0

    o_ref[0, 0] += s


def _tc_reduce(dots):
    dots2d = jnp.reshape(dots, (_TC_ROWS, _TC_COLS))
    return pl.pallas_call(
        _tc_reduce_body,
        grid=(_TC_GRID,),
        in_specs=[pl.BlockSpec((_TC_BLK, _TC_COLS), lambda i: (i, 0))],
        out_specs=pl.BlockSpec(memory_space=pltpu.SMEM),
        out_shape=jax.ShapeDtypeStruct((1, 1), jnp.float32),
    )(dots2d)


def kernel(input_labels, out_labels, noise_indices, node_emb, ctx_emb):
    inp_idx2d = jnp.reshape(input_labels.astype(jnp.int32),
                            (BATCH // CHUNK_WIN, CHUNK_WIN))
    out_idx2d = jnp.reshape(out_labels.astype(jnp.int32), (BW // 80, 80))
    noise_idx2d = jnp.reshape(noise_indices.astype(jnp.int32),
                              (BW * NEG // 100, 100))
    winrow = jnp.asarray(np.arange(CHUNK_PAIRS) // WINDOW, dtype=jnp.int32)
    dots = _sc_dots(node_emb, ctx_emb, inp_idx2d, out_idx2d, noise_idx2d,
                    winrow)
    total = _tc_reduce(dots)
    return -total[0, 0] / BATCH


# speedup vs baseline: 2.2073x; 2.0848x over previous
import functools

import jax
import jax.numpy as jnp
import numpy as np
from jax import lax
from jax.experimental import pallas as pl
from jax.experimental.pallas import tpu as pltpu
from jax.experimental.pallas import tpu_sc as plsc

VOCAB = 1000000
DIM = 64
BATCH = 16384
WINDOW = 20
NEG = 5
BW = BATCH * WINDOW
PAD = 8

NW = 32
WIN_PER_SUB = BATCH // NW
CHUNK_WIN = 4
CHUNK_PAIRS = CHUNK_WIN * WINDOW
CHUNK_NOISE = CHUNK_PAIRS * NEG
BLK_CHUNKS = 16
BLK_WIN = CHUNK_WIN * BLK_CHUNKS
NBLK = WIN_PER_SUB // BLK_WIN
NBLK_TOTAL = NW * NBLK
GROUPS = CHUNK_PAIRS // 16
DOTS_CHUNK = CHUNK_PAIRS * PAD


def _sc_dots(node_emb, ctx_emb, inp_idx3, out_idx3, noise_idx3, winrow):
    mesh = plsc.VectorSubcoreMesh(core_axis_name="c", subcore_axis_name="s")

    @functools.partial(
        pl.kernel,
        out_type=jax.ShapeDtypeStruct((BW * PAD,), jnp.float32),
        mesh=mesh,
        compiler_params=pltpu.CompilerParams(needs_layout_passes=False,
                                             use_tc_tiling_on_sc=False),
        scratch_types=[
            pltpu.VMEM((BLK_WIN,), jnp.int32),
            pltpu.VMEM((BLK_CHUNKS, CHUNK_PAIRS), jnp.int32),
            pltpu.VMEM((BLK_CHUNKS * 4, 100), jnp.int32),
            pltpu.VMEM((CHUNK_PAIRS,), jnp.int32),
            pltpu.VMEM((BLK_WIN, DIM), jnp.float32),
            pltpu.VMEM((CHUNK_PAIRS, DIM), jnp.float32),
            pltpu.VMEM((CHUNK_PAIRS, DIM), jnp.float32),
            pltpu.VMEM((CHUNK_NOISE, DIM), jnp.float32),
            pltpu.VMEM((CHUNK_NOISE, DIM), jnp.float32),
            pltpu.VMEM((DOTS_CHUNK,), jnp.float32),
            pltpu.VMEM((DOTS_CHUNK,), jnp.float32),
            pltpu.SemaphoreType.DMA,
            pltpu.SemaphoreType.DMA,
            pltpu.SemaphoreType.DMA,
        ],
    )
    def k(node_hbm, ctx_hbm, inp_hbm, out_hbm, noise_hbm, winrow_hbm,
          dots_hbm, inp_idx_v, out_idx_v, noise_idx_v, winrow_v,
          inp_rows, out_rows0, out_rows1, noise_rows0, noise_rows1,
          dots0, dots1, sem_g0, sem_g1, sem_wb):
        wid = lax.axis_index("s") * 2 + lax.axis_index("c")
        pltpu.sync_copy(winrow_hbm, winrow_v)
        lam = lax.iota(jnp.int32, 16)
        out_rows = (out_rows0, out_rows1)
        noise_rows = (noise_rows0, noise_rows1)
        dots_v = (dots0, dots1)
        sem_g = (sem_g0, sem_g1)

        def issue_gathers(cc, slot):
            hs = [pltpu.async_copy(ctx_hbm.at[out_idx_v.at[cc]],
                                   out_rows[slot], sem_g[slot])]
            for j in range(4):
                hs.append(pltpu.async_copy(
                    ctx_hbm.at[noise_idx_v.at[cc * 4 + j]],
                    noise_rows[slot].at[pl.ds(j * 100, 100)], sem_g[slot]))
            return hs

        DUNROLL = 8

        def compute_chunk(cc, slot):
            def group_body(g, carry):
                pair16 = g * 16 + lam
                win_v = cc * CHUNK_WIN + winrow_v[pl.ds(g * 16, 16)]
                pair5 = pair16 * NEG

                def d_body(i, accs):
                    accs = list(accs)
                    for u in range(DUNROLL):
                        dv = (lam + i * DUNROLL + u) & (DIM - 1)
                        a = plsc.load_gather(inp_rows, [win_v, dv])
                        o = plsc.load_gather(out_rows[slot], [pair16, dv])
                        accs[0] = accs[0] + a * o
                        for n in range(NEG):
                            x = plsc.load_gather(noise_rows[slot],
                                                 [pair5 + n, dv])
                            accs[1 + n] = accs[1 + n] - a * x
                    return tuple(accs)

                zero = jnp.zeros((16,), jnp.float32)
                accs = lax.fori_loop(0, DIM // DUNROLL, d_body, (zero,) * 6)
                base8 = pair16 * PAD
                for t in range(6):
                    plsc.store_scatter(dots_v[slot], [base8 + t], accs[t])
                return carry

            lax.fori_loop(0, GROUPS, group_body, 0)

        def block_body(b, carry):
            g_b = wid * NBLK + b
            pltpu.sync_copy(inp_hbm.at[g_b], inp_idx_v)
            pltpu.sync_copy(out_hbm.at[g_b], out_idx_v)
            pltpu.sync_copy(noise_hbm.at[g_b], noise_idx_v)
            h_inp = pltpu.async_copy(node_hbm.at[inp_idx_v], inp_rows,
                                     sem_g0)
            pending = {0: [h_inp] + issue_gathers(0, 0), 1: []}
            wb = {0: None, 1: None}
            for cc in range(BLK_CHUNKS):
                slot = cc % 2
                if cc + 1 < BLK_CHUNKS:
                    pending[1 - slot] = issue_gathers(cc + 1, 1 - slot)
                for h in pending[slot]:
                    h.wait()
                if wb[slot] is not None:
                    wb[slot].wait()
                compute_chunk(cc, slot)
                wb[slot] = pltpu.async_copy(
                    dots_v[slot],
                    dots_hbm.at[pl.ds((g_b * BLK_CHUNKS + cc) * DOTS_CHUNK,
                                      DOTS_CHUNK)],
                    sem_wb)
            wb[0].wait()
            wb[1].wait()
            return carry

        lax.fori_loop(0, NBLK, block_body, 0)

    return k(node_emb, ctx_emb, inp_idx3, out_idx3, noise_idx3, winrow)


_TC_ROWS = 2560
_TC_COLS = 1024
_TC_BLK = 256
_TC_GRID = _TC_ROWS // _TC_BLK


def _tc_reduce_body(x_ref, o_ref):
    i = pl.program_id(0)
    x = x_ref[...]
    z = jnp.minimum(x, 0.0) - jnp.log1p(jnp.exp(-jnp.abs(x)))
    col = lax.broadcasted_iota(jnp.int32, (_TC_BLK, _TC_COLS), 1)
    z = jnp.where((col % PAD) < 6, z, 0.0)
    s = jnp.sum(z)

    @pl.when(i == 0)
    def _():
        o_ref[0, 0] = 0.0

    o_ref[0, 0] += s


def _tc_reduce(dots):
    dots2d = jnp.reshape(dots, (_TC_ROWS, _TC_COLS))
    return pl.pallas_call(
        _tc_reduce_body,
        grid=(_TC_GRID,),
        in_specs=[pl.BlockSpec((_TC_BLK, _TC_COLS), lambda i: (i, 0))],
        out_specs=pl.BlockSpec(memory_space=pltpu.SMEM),
        out_shape=jax.ShapeDtypeStruct((1, 1), jnp.float32),
    )(dots2d)


def kernel(input_labels, out_labels, noise_indices, node_emb, ctx_emb):
    inp_idx3 = jnp.reshape(input_labels.astype(jnp.int32),
                           (NBLK_TOTAL, BLK_WIN))
    out_idx3 = jnp.reshape(out_labels.astype(jnp.int32),
                           (NBLK_TOTAL, BLK_CHUNKS, CHUNK_PAIRS))
    noise_idx3 = jnp.reshape(noise_indices.astype(jnp.int32),
                             (NBLK_TOTAL, BLK_CHUNKS * 4, 100))
    winrow = jnp.asarray(np.arange(CHUNK_PAIRS) // WINDOW, dtype=jnp.int32)
    dots = _sc_dots(node_emb, ctx_emb, inp_idx3, out_idx3, noise_idx3,
                    winrow)
    total = _tc_reduce(dots)
    return -total[0, 0] / BATCH
